# baseline (speedup 1.0000x reference)
import functools
import math

import jax
import jax.numpy as jnp
from jax import lax
from jax.experimental import pallas as pl
from jax.experimental.pallas import tpu as pltpu
from jax.experimental.pallas import tpu_sc as plsc

D_MODEL = 64
SCALE = math.sqrt(D_MODEL)
LANES = 16
IDXW = 128
GRP = 256
SUB = GRP // IDXW
NG = 4
NS = 2
DT = 8
BT = 128


@functools.cache
def _make_relayout(V: int):
    info = plsc.get_sparse_core_info()
    num_workers = info.num_cores * info.num_subcores
    CH = 128
    n_full = V // CH
    tail = V - n_full * CH
    base_n, extra = divmod(n_full, num_workers)
    mesh = plsc.VectorSubcoreMesh(core_axis_name="c", subcore_axis_name="s")
    NB = 2

    @functools.partial(
        pl.kernel,
        mesh=mesh,
        out_type=jax.ShapeDtypeStruct((V, 2 * D_MODEL), jnp.float32),
        compiler_params=pltpu.CompilerParams(needs_layout_passes=False),
        scratch_types=(
            [pltpu.VMEM((D_MODEL, CH), jnp.float32) for _ in range(NB)]
            + [pltpu.VMEM((CH, 2 * D_MODEL), jnp.float32) for _ in range(NB)]
            + [pltpu.VMEM((D_MODEL, CH // 2), jnp.float32)]
            + [pltpu.SemaphoreType.DMA for _ in range(2 * NB)]
        ),
    )
    def k(tn_hbm, out_hbm, *bufs_and_sems):
        cbuf = bufs_and_sems[:NB]
        obuf = bufs_and_sems[NB:2 * NB]
        tailc = bufs_and_sems[2 * NB]
        lsem = bufs_and_sems[2 * NB + 1:3 * NB + 1]
        ssem = bufs_and_sems[3 * NB + 1:]

        wid = lax.axis_index("s") * info.num_cores + lax.axis_index("c")
        my_n = jnp.where(wid < extra, base_n + 1, base_n)

        def v0_of(i):
            c = wid + i * num_workers
            return pl.multiple_of(c * CH, CH)

        def fire_load(i, b):
            pltpu.async_copy(
                tn_hbm.at[:, pl.ds(v0_of(i), CH)], cbuf[b], lsem[b]
            )

        def wait_load(i, b):
            pltpu.make_async_copy(
                tn_hbm.at[:, pl.ds(v0_of(i), CH)], cbuf[b], lsem[b]
            ).wait()

        def fire_store(i, b):
            pltpu.async_copy(
                obuf[b], out_hbm.at[pl.ds(v0_of(i), CH)], ssem[b]
            )

        def wait_store(i, b):
            pltpu.make_async_copy(
                obuf[b], out_hbm.at[pl.ds(v0_of(i), CH)], ssem[b]
            ).wait()

        lane_iota = lax.iota(jnp.int32, LANES)

        def transpose(b):
            @plsc.parallel_loop(0, D_MODEL // LANES)
            def _(db):
                rows_d = lane_iota + db * LANES
                for vb in range(CH // LANES):
                    for s in range(LANES):
                        perm = (lane_iota + s) & (LANES - 1)
                        vv = perm + (vb * LANES)
                        v = plsc.load_gather(cbuf[b], [rows_d, vv])
                        plsc.store_scatter(obuf[b], [vv, rows_d], v * SCALE)

        for b in range(NB):
            @pl.when(b < my_n)
            def _():
                fire_load(b, b)

        def body(i, carry):
            for par in range(NB):
                ii = i * NB + par

                @pl.when(ii < my_n)
                def _():
                    wait_load(ii, par)

                    @pl.when(ii >= NB)
                    def _():
                        wait_store(ii - NB, par)

                    transpose(par)
                    fire_store(ii, par)

                    @pl.when(ii + NB < my_n)
                    def _():
                        fire_load(ii + NB, par)

            return carry

        assert base_n >= NB
        lax.fori_loop(0, (base_n + 1 + NB - 1) // NB + 1, body, 0)
        for par in range(NB):
            wait_store(my_n - NB + par, par)

        if tail:
            @pl.when(wid == num_workers - 1)
            def _():
                tv0 = n_full * CH
                pltpu.sync_copy(tn_hbm.at[:, pl.ds(tv0, tail)], tailc)

                @plsc.parallel_loop(0, D_MODEL // LANES)
                def _(db):
                    rows_d = lane_iota + db * LANES
                    for vb in range(tail // LANES):
                        for s in range(LANES):
                            perm = (lane_iota + s) & (LANES - 1)
                            vv = perm + (vb * LANES)
                            v = plsc.load_gather(tailc, [rows_d, vv])
                            plsc.store_scatter(obuf[0], [vv, rows_d], v * SCALE)

                pltpu.sync_copy(
                    obuf[0].at[pl.ds(0, tail)], out_hbm.at[pl.ds(tv0, tail)]
                )

    return k


@functools.cache
def _make(T: int, B: int):
    info = plsc.get_sparse_core_info()
    num_workers = info.num_cores * info.num_subcores
    n_idx = T * B
    per_w = n_idx // num_workers
    n_groups = per_w // GRP
    rows_per_w = per_w // IDXW
    groups_per_row = B // GRP
    bt_per_grp = GRP // BT
    assert n_idx % num_workers == 0 and per_w % GRP == 0 and n_groups % NG == 0
    mesh = plsc.VectorSubcoreMesh(core_axis_name="c", subcore_axis_name="s")

    @functools.partial(
        pl.kernel,
        mesh=mesh,
        out_type=jax.ShapeDtypeStruct(
            (T, D_MODEL // DT, B // BT, DT, BT), jnp.float32
        ),
        compiler_params=pltpu.CompilerParams(
            use_tc_tiling_on_sc=False, needs_layout_passes=False
        ),
        scratch_types=(
            [pltpu.VMEM((rows_per_w, IDXW), jnp.int32)]
            + [pltpu.VMEM((GRP, D_MODEL), jnp.float32) for _ in range(NG)]
            + [pltpu.VMEM((BT, BT), jnp.float32) for _ in range(NS)]
            + [pltpu.SemaphoreType.DMA for _ in range(NG + NS)]
        ),
    )
    def k(idx_hbm, table_hbm, out_hbm, idx_v, *bufs_and_sems):
        gbuf = bufs_and_sems[:NG]
        tbuf = bufs_and_sems[NG:NG + NS]
        gsem = bufs_and_sems[NG + NS:2 * NG + NS]
        ssem = bufs_and_sems[2 * NG + NS:]

        wid = lax.axis_index("s") * info.num_cores + lax.axis_index("c")
        g0w = wid * n_groups

        pltpu.sync_copy(idx_hbm.at[pl.ds(wid * rows_per_w, rows_per_w)], idx_v)

        @plsc.parallel_loop(0, rows_per_w, unroll=4)
        def _(r):
            for j in range(IDXW // LANES):
                sl = pl.ds(j * LANES, LANES)
                idx_v[r, sl] = idx_v[r, sl] * 2

        def fire_gather(gi, b):
            for j in range(SUB):
                pltpu.async_copy(
                    table_hbm.at[idx_v.at[gi * SUB + j]],
                    gbuf[b].at[pl.ds(j * IDXW, IDXW)],
                    gsem[b],
                )

        def wait_gather(gi, b):
            for j in range(SUB):
                pltpu.make_async_copy(
                    table_hbm.at[idx_v.at[gi * SUB + j]],
                    gbuf[b].at[pl.ds(j * IDXW, IDXW)],
                    gsem[b],
                ).wait()

        def store_copies(gi, bs):
            g = g0w + gi
            t = g // groups_per_row
            bc = g - t * groups_per_row
            for dt in range(D_MODEL // DT):
                for bt2 in range(bt_per_grp):
                    yield (
                        tbuf[bs].at[pl.ds(bt2 * D_MODEL + dt * DT, DT)],
                        out_hbm.at[t, dt, bc * bt_per_grp + bt2],
                    )

        def fire_store(gi, bs):
            for src, dst in store_copies(gi, bs):
                pltpu.async_copy(src, dst, ssem[bs])

        def wait_store(gi, bs):
            for src, dst in store_copies(gi, bs):
                pltpu.make_async_copy(src, dst, ssem[bs]).wait()

        lane_iota = lax.iota(jnp.int32, LANES)

        def transpose_scale(b, bs):
            @plsc.parallel_loop(0, GRP // LANES)
            def _(b16):
                b0 = b16 * LANES
                rows = lane_iota + b0
                bt2 = b0 // BT
                bl0 = b0 - bt2 * BT
                bls = lane_iota + bl0
                for j in range(D_MODEL // LANES):
                    for s in range(LANES):
                        perm = (lane_iota + s) & (LANES - 1)
                        cols = perm + (j * LANES)
                        v = plsc.load_gather(gbuf[b], [rows, cols])
                        trow = cols + (bt2 * D_MODEL)
                        plsc.store_scatter(tbuf[bs], [trow, bls], v)

        for b in range(NG):
            fire_gather(b, b)

        def outer(i0, carry):
            for b in range(NG):
                gi = i0 * NG + b
                bs = b % NS
                wait_gather(gi, b)

                @pl.when(jnp.logical_or(i0 > 0, b >= NS))
                def _():
                    wait_store(gi - NS, bs)

                transpose_scale(b, bs)
                fire_store(gi, bs)

                @pl.when(i0 < n_groups // NG - 1)
                def _():
                    fire_gather(gi + NG, b)

            return carry

        lax.fori_loop(0, n_groups // NG, outer, 0)
        for b in range(NS):
            wait_store(n_groups - NS + b, (n_groups - NS + b) % NS)

    return k


def kernel(x, table):
    bsz, T = x.shape
    V = table.shape[0]
    xt = x.T.reshape((bsz * T) // IDXW, IDXW)
    relaid = _make_relayout(V)(table.T)
    tpad = relaid.reshape(2 * V, D_MODEL)
    out6 = _make(T, bsz)(xt, tpad)
    return out6.transpose(2, 4, 0, 1, 3).reshape(bsz, T, D_MODEL)


# speedup vs baseline: 1.3479x; 1.3479x over previous
import functools
import math

import jax
import jax.numpy as jnp
from jax import lax
from jax.experimental import pallas as pl
from jax.experimental.pallas import tpu as pltpu
from jax.experimental.pallas import tpu_sc as plsc

D_MODEL = 64
SCALE = math.sqrt(D_MODEL)
LANES = 16
IDXW = 128
GRP = 256
SUB = GRP // IDXW
NG = 4
NS = 2
DT = 8
BT = 128


@functools.cache
def _make(T: int, B: int):
    info = plsc.get_sparse_core_info()
    num_workers = info.num_cores * info.num_subcores
    n_idx = T * B
    per_w = n_idx // num_workers
    n_groups = per_w // GRP
    rows_per_w = per_w // IDXW
    groups_per_row = B // GRP
    bt_per_grp = GRP // BT
    assert n_idx % num_workers == 0 and per_w % GRP == 0 and n_groups % NG == 0
    mesh = plsc.VectorSubcoreMesh(core_axis_name="c", subcore_axis_name="s")

    @functools.partial(
        pl.kernel,
        mesh=mesh,
        out_type=jax.ShapeDtypeStruct(
            (T, D_MODEL // DT, B // BT, DT, BT), jnp.float32
        ),
        compiler_params=pltpu.CompilerParams(
            use_tc_tiling_on_sc=False, needs_layout_passes=False
        ),
        scratch_types=(
            [pltpu.VMEM((rows_per_w, IDXW), jnp.int32)]
            + [pltpu.VMEM((GRP, D_MODEL), jnp.float32) for _ in range(NG)]
            + [pltpu.VMEM((BT, BT), jnp.float32) for _ in range(NS)]
            + [pltpu.SemaphoreType.DMA for _ in range(NG + NS)]
        ),
    )
    def k(idx_hbm, table_hbm, out_hbm, idx_v, *bufs_and_sems):
        gbuf = bufs_and_sems[:NG]
        tbuf = bufs_and_sems[NG:NG + NS]
        gsem = bufs_and_sems[NG + NS:2 * NG + NS]
        ssem = bufs_and_sems[2 * NG + NS:]

        wid = lax.axis_index("s") * info.num_cores + lax.axis_index("c")
        g0w = wid * n_groups

        pltpu.sync_copy(idx_hbm.at[pl.ds(wid * rows_per_w, rows_per_w)], idx_v)

        @plsc.parallel_loop(0, rows_per_w, unroll=4)
        def _(r):
            for j in range(IDXW // LANES):
                sl = pl.ds(j * LANES, LANES)
                idx_v[r, sl] = idx_v[r, sl] * 2

        def fire_gather(gi, b):
            for j in range(SUB):
                pltpu.async_copy(
                    table_hbm.at[idx_v.at[gi * SUB + j]],
                    gbuf[b].at[pl.ds(j * IDXW, IDXW)],
                    gsem[b],
                )

        def wait_gather(gi, b):
            for j in range(SUB):
                pltpu.make_async_copy(
                    table_hbm.at[idx_v.at[gi * SUB + j]],
                    gbuf[b].at[pl.ds(j * IDXW, IDXW)],
                    gsem[b],
                ).wait()

        def store_copies(gi, bs):
            g = g0w + gi
            t = g // groups_per_row
            bc = g - t * groups_per_row
            for dt in range(D_MODEL // DT):
                for bt2 in range(bt_per_grp):
                    yield (
                        tbuf[bs].at[pl.ds(bt2 * D_MODEL + dt * DT, DT)],
                        out_hbm.at[t, dt, bc * bt_per_grp + bt2],
                    )

        def fire_store(gi, bs):
            for src, dst in store_copies(gi, bs):
                pltpu.async_copy(src, dst, ssem[bs])

        def wait_store(gi, bs):
            for src, dst in store_copies(gi, bs):
                pltpu.make_async_copy(src, dst, ssem[bs]).wait()

        lane_iota = lax.iota(jnp.int32, LANES)

        def transpose_scale(b, bs):
            @plsc.parallel_loop(0, GRP // LANES)
            def _(b16):
                b0 = b16 * LANES
                rows = lane_iota + b0
                bt2 = b0 // BT
                bl0 = b0 - bt2 * BT
                bls = lane_iota + bl0
                for j in range(D_MODEL // LANES):
                    for s in range(LANES):
                        perm = (lane_iota + s) & (LANES - 1)
                        cols = perm + (j * LANES)
                        v = plsc.load_gather(gbuf[b], [rows, cols])
                        trow = cols + (bt2 * D_MODEL)
                        plsc.store_scatter(tbuf[bs], [trow, bls], v * SCALE)

        for b in range(NG):
            fire_gather(b, b)

        def outer(i0, carry):
            for b in range(NG):
                gi = i0 * NG + b
                bs = b % NS
                wait_gather(gi, b)

                @pl.when(jnp.logical_or(i0 > 0, b >= NS))
                def _():
                    wait_store(gi - NS, bs)

                transpose_scale(b, bs)
                fire_store(gi, bs)

                @pl.when(i0 < n_groups // NG - 1)
                def _():
                    fire_gather(gi + NG, b)

            return carry

        lax.fori_loop(0, n_groups // NG, outer, 0)
        for b in range(NS):
            wait_store(n_groups - NS + b, (n_groups - NS + b) % NS)

    return k


def kernel(x, table):
    bsz, T = x.shape
    xt = x.T.reshape((bsz * T) // IDXW, IDXW)
    tpad = jnp.pad(table, ((0, 0), (0, D_MODEL))).reshape(-1, D_MODEL)
    out6 = _make(T, bsz)(xt, tpad)
    return out6.transpose(2, 4, 0, 1, 3).reshape(bsz, T, D_MODEL)
